# Initial kernel scaffold; baseline (speedup 1.0000x reference)
import jax
import jax.numpy as jnp
from jax.experimental import pallas as pl


def kernel(nodes, edges, senders, receivers, enc_node_W0, enc_node_b0, enc_node_W1, enc_node_b1, enc_edge_W0, enc_edge_b0, enc_edge_W1, enc_edge_b1, W_message, W_node, nodeMLP_W0, nodeMLP_b0, nodeMLP_W1, nodeMLP_b1, ln_scale, ln_bias, dec_W0, dec_b0, dec_W1, dec_b1):
    raise NotImplementedError("write your pallas kernel here")


# speedup vs baseline: 4.8796x; 4.8796x over previous
import functools

import jax
import jax.numpy as jnp
from jax import lax
from jax.experimental import pallas as pl
from jax.experimental.pallas import tpu as pltpu
from jax.experimental.pallas import tpu_sc as plsc

_NC = 2
_NS = 16
_NW = _NC * _NS
_CB = 80
_D = 32
_DW = 128
_NP = 10240


def _sc_mesh():
    return plsc.VectorSubcoreMesh(core_axis_name="c", subcore_axis_name="s")


def _build_gather_segsum(n_edges):
    epw = n_edges // _NW
    nch = epw // _CB
    rpt = _NP // _NS

    @functools.partial(
        pl.kernel,
        mesh=_sc_mesh(),
        out_type=jax.ShapeDtypeStruct((_NC, _NP, _DW), jnp.float32),
        scratch_types=[
            pltpu.VMEM((nch, _CB), jnp.int32),
            pltpu.VMEM((nch, _CB), jnp.int32),
            pltpu.VMEM((_CB, _DW), jnp.float32),
            pltpu.VMEM_SHARED((_NP, _DW), jnp.float32),
            pltpu.SemaphoreType.DMA,
        ],
    )
    def k(m_hbm, send_hbm, recv_hbm, z_hbm, out_hbm, sidx, ridx, buf, agg_sh, sem):
        cid = lax.axis_index("c")
        sid = lax.axis_index("s")
        wid = sid * _NC + cid
        pltpu.sync_copy(z_hbm, agg_sh.at[pl.ds(sid * rpt, rpt)])
        pltpu.sync_copy(send_hbm.at[wid], sidx)
        pltpu.sync_copy(recv_hbm.at[wid], ridx)
        plsc.subcore_barrier()

        def body(j, carry):
            pltpu.async_copy(m_hbm.at[sidx.at[j]], buf, sem).wait()
            pltpu.sync_copy(buf, agg_sh.at[ridx.at[j]], add=True)
            return carry

        lax.fori_loop(0, nch, body, 0)
        plsc.subcore_barrier()
        pltpu.sync_copy(agg_sh.at[pl.ds(sid * rpt, rpt)],
                        out_hbm.at[cid, pl.ds(sid * rpt, rpt)])

    return k


def _build_linear_segsum(n_edges):
    epw = n_edges // _NW
    nch = epw // _CB
    rpt = _NP // _NS

    @functools.partial(
        pl.kernel,
        mesh=_sc_mesh(),
        out_type=jax.ShapeDtypeStruct((_NC, _NP, _DW), jnp.float32),
        scratch_types=[
            pltpu.VMEM((nch, _CB), jnp.int32),
            pltpu.VMEM((_CB, _DW), jnp.float32),
            pltpu.VMEM_SHARED((_NP, _DW), jnp.float32),
            pltpu.SemaphoreType.DMA,
        ],
    )
    def k(x_hbm, recv_hbm, z_hbm, out_hbm, ridx, buf, agg_sh, sem):
        cid = lax.axis_index("c")
        sid = lax.axis_index("s")
        wid = sid * _NC + cid
        base = wid * epw
        pltpu.sync_copy(z_hbm, agg_sh.at[pl.ds(sid * rpt, rpt)])
        pltpu.sync_copy(recv_hbm.at[wid], ridx)
        plsc.subcore_barrier()

        def body(j, carry):
            pltpu.async_copy(x_hbm.at[pl.ds(base + j * _CB, _CB)], buf, sem).wait()
            pltpu.sync_copy(buf, agg_sh.at[ridx.at[j]], add=True)
            return carry

        lax.fori_loop(0, nch, body, 0)
        plsc.subcore_barrier()
        pltpu.sync_copy(agg_sh.at[pl.ds(sid * rpt, rpt)],
                        out_hbm.at[cid, pl.ds(sid * rpt, rpt)])

    return k


def _full(shape):
    return pl.BlockSpec(shape, lambda i: tuple(0 for _ in shape))


def _enc_nodes(nodes, w0, b0, w1, b1, wmt_w):
    n, df = nodes.shape
    bn = 1000

    def body(x_ref, w0r, b0r, w1r, b1r, wmtr, hn_ref, m_ref):
        h = jnp.maximum(x_ref[...] @ w0r[...] + b0r[...], 0.0)
        hn = h @ w1r[...] + b1r[...]
        hn_ref[...] = hn
        m_ref[...] = hn @ wmtr[...]

    return pl.pallas_call(
        body,
        grid=(n // bn,),
        in_specs=[
            pl.BlockSpec((bn, df), lambda i: (i, 0)),
            _full((df, 64)), _full((1, 64)), _full((64, _D)), _full((1, _D)),
            _full((_D, _DW)),
        ],
        out_specs=[pl.BlockSpec((bn, _D), lambda i: (i, 0)),
                   pl.BlockSpec((bn, _DW), lambda i: (i, 0))],
        out_shape=[jax.ShapeDtypeStruct((n, _D), jnp.float32),
                   jax.ShapeDtypeStruct((n, _DW), jnp.float32)],
    )(nodes, w0, b0.reshape(1, 64), w1, b1.reshape(1, _D), wmt_w)


def _enc_edges(edges, w0, b0, w1, b1, wmb_w):
    e, de = edges.shape
    be = 4000

    def body(x_ref, w0r, b0r, w1r, b1r, wmbr, out_ref):
        h = jnp.maximum(x_ref[...] @ w0r[...] + b0r[...], 0.0)
        wc = w1r[...] @ wmbr[...]
        bc = b1r[...] @ wmbr[...]
        out_ref[...] = h @ wc + bc

    return pl.pallas_call(
        body,
        grid=(e // be,),
        in_specs=[
            pl.BlockSpec((be, de), lambda i: (i, 0)),
            _full((de, 64)), _full((1, 64)), _full((64, _D)), _full((1, _D)),
            _full((_D, _DW)),
        ],
        out_specs=pl.BlockSpec((be, _DW), lambda i: (i, 0)),
        out_shape=jax.ShapeDtypeStruct((e, _DW), jnp.float32),
    )(edges, w0, b0.reshape(1, 64), w1, b1.reshape(1, _D), wmb_w)


def _step(hn, parts, agge, w0a, w0b, b0, w1, b1, wn, lns, lnb, wmt_w, last,
          dw0, db0, dw1, db1):
    n, _ = hn.shape
    bn = 1000
    df = dw1.shape[1]

    def node_update(hn_ref, p_ref, pe_ref, w0ar, w0br, b0r, w1r, b1r, wnr,
                    sr, br):
        agg = (p_ref[0, :, :_D] + p_ref[1, :, :_D]
               + pe_ref[0, :, :_D] + pe_ref[1, :, :_D])
        t = jnp.maximum(hn_ref[...] @ w0ar[...] + agg @ w0br[...] + b0r[...],
                        0.0) @ w1r[...] + b1r[...]
        x = hn_ref[...] @ wnr[...] + t
        mu = jnp.mean(x, axis=-1, keepdims=True)
        var = jnp.mean((x - mu) ** 2, axis=-1, keepdims=True)
        return (x - mu) * lax.rsqrt(var + 1e-6) * sr[...] + br[...]

    common_specs = [
        pl.BlockSpec((bn, _D), lambda i: (i, 0)),
        pl.BlockSpec((_NC, bn, _DW), lambda i: (0, i, 0)),
        pl.BlockSpec((_NC, bn, _DW), lambda i: (0, i, 0)),
        _full((_D, _D)), _full((_D, _D)), _full((1, _D)), _full((_D, _D)),
        _full((1, _D)), _full((_D, _D)), _full((1, _D)), _full((1, _D)),
    ]
    common_args = (hn, parts, agge, w0a, w0b, b0.reshape(1, _D), w1,
                   b1.reshape(1, _D), wn, lns.reshape(1, _D),
                   lnb.reshape(1, _D))

    if not last:
        def body(hn_ref, p_ref, pe_ref, w0ar, w0br, b0r, w1r, b1r, wnr, sr,
                 br, wmtr, hn_out, m_out):
            y = node_update(hn_ref, p_ref, pe_ref, w0ar, w0br, b0r, w1r, b1r,
                            wnr, sr, br)
            hn_out[...] = y
            m_out[...] = y @ wmtr[...]

        return pl.pallas_call(
            body,
            grid=(n // bn,),
            in_specs=common_specs + [_full((_D, _DW))],
            out_specs=[pl.BlockSpec((bn, _D), lambda i: (i, 0)),
                       pl.BlockSpec((bn, _DW), lambda i: (i, 0))],
            out_shape=[jax.ShapeDtypeStruct((n, _D), jnp.float32),
                       jax.ShapeDtypeStruct((n, _DW), jnp.float32)],
        )(*common_args, wmt_w)

    def body(hn_ref, p_ref, pe_ref, w0ar, w0br, b0r, w1r, b1r, wnr, sr, br,
             dw0r, db0r, dw1r, db1r, out_ref):
        y = node_update(hn_ref, p_ref, pe_ref, w0ar, w0br, b0r, w1r, b1r,
                        wnr, sr, br)
        d = jnp.maximum(y @ dw0r[...] + db0r[...], 0.0) @ dw1r[...] + db1r[...]
        out_ref[...] = d

    return pl.pallas_call(
        body,
        grid=(n // bn,),
        in_specs=common_specs + [_full((_D, 64)), _full((1, 64)),
                                 _full((64, df)), _full((1, df))],
        out_specs=pl.BlockSpec((bn, df), lambda i: (i, 0)),
        out_shape=jax.ShapeDtypeStruct((n, df), jnp.float32),
    )(*common_args, dw0, db0.reshape(1, 64), dw1, db1.reshape(1, df))


def kernel(nodes, edges, senders, receivers,
           enc_node_W0, enc_node_b0, enc_node_W1, enc_node_b1,
           enc_edge_W0, enc_edge_b0, enc_edge_W1, enc_edge_b1,
           W_message, W_node,
           nodeMLP_W0, nodeMLP_b0, nodeMLP_W1, nodeMLP_b1,
           ln_scale, ln_bias,
           dec_W0, dec_b0, dec_W1, dec_b1):
    n, _ = nodes.shape
    e, _ = edges.shape
    epw = e // _NW
    nch = epw // _CB

    senders = senders.astype(jnp.int32).reshape(_NW, nch, _CB)
    receivers = receivers.astype(jnp.int32).reshape(_NW, nch, _CB)
    zeros = jnp.zeros((_NP // _NS, _DW), jnp.float32)

    wmt_w = jnp.pad(W_message[:_D], ((0, 0), (0, _DW - _D)))
    wmb_w = jnp.pad(W_message[_D:], ((0, 0), (0, _DW - _D)))
    w0a = nodeMLP_W0[:_D]
    w0b = nodeMLP_W0[_D:]

    gather_segsum = _build_gather_segsum(e)
    linear_segsum = _build_linear_segsum(e)

    hn, m = _enc_nodes(nodes, enc_node_W0, enc_node_b0, enc_node_W1,
                       enc_node_b1, wmt_w)
    msg_e = _enc_edges(edges, enc_edge_W0, enc_edge_b0, enc_edge_W1,
                       enc_edge_b1, wmb_w)
    agge = linear_segsum(msg_e, receivers, zeros)

    for step in range(5):
        parts = gather_segsum(m, senders, receivers, zeros)
        out = _step(hn, parts, agge, w0a, w0b, nodeMLP_b0, nodeMLP_W1,
                    nodeMLP_b1, W_node, ln_scale, ln_bias, wmt_w,
                    step == 4, dec_W0, dec_b0, dec_W1, dec_b1)
        if step < 4:
            hn, m = out
        else:
            return out


# speedup vs baseline: 7.1028x; 1.4556x over previous
import functools

import jax
import jax.numpy as jnp
from jax import lax
from jax.experimental import pallas as pl
from jax.experimental.pallas import tpu as pltpu
from jax.experimental.pallas import tpu_sc as plsc

_NC = 2
_NS = 16
_NW = _NC * _NS
_CB = 80
_D = 32
_DW = 128
_NP = 10240


def _sc_mesh():
    return plsc.VectorSubcoreMesh(core_axis_name="c", subcore_axis_name="s")


def _build_segsum(n_edges, gather):
    epw = n_edges // _NW
    nch = epw // _CB
    rpt = _NP // _NS

    scratch = [
        pltpu.VMEM((nch, _CB), jnp.int32),
        pltpu.VMEM((2, _CB), jnp.int32),
        pltpu.VMEM((2, _CB), jnp.int32),
        pltpu.VMEM((_CB, _DW), jnp.float32),
        pltpu.VMEM((_CB, _DW), jnp.float32),
        pltpu.VMEM_SHARED((_NP, _DW), jnp.float32),
        pltpu.SemaphoreType.DMA,
        pltpu.SemaphoreType.DMA,
    ]

    @functools.partial(
        pl.kernel,
        mesh=_sc_mesh(),
        out_type=jax.ShapeDtypeStruct((_NC, _NP, _DW), jnp.float32),
        scratch_types=scratch,
    )
    def k(x_hbm, idx_hbm, z_hbm, out_hbm,
          staged, sidx, ridx, buf0, buf1, agg_sh, sem0, sem1):
        cid = lax.axis_index("c")
        sid = lax.axis_index("s")
        wid = sid * _NC + cid
        base = wid * epw
        pltpu.sync_copy(z_hbm, agg_sh.at[pl.ds(sid * rpt, rpt)])
        pltpu.sync_copy(idx_hbm.at[wid], staged)
        plsc.subcore_barrier()

        bufs = (buf0, buf1)
        sems = (sem0, sem1)

        def prep(j, b):
            for t in range(_CB // 16):
                sl = pl.ds(t * 16, 16)
                p = staged[j, sl]
                if gather:
                    sidx[b, sl] = lax.bitwise_and(p, 16383)
                    ridx[b, sl] = lax.shift_right_logical(p, 14)
                else:
                    ridx[b, sl] = p
            if gather:
                pltpu.async_copy(x_hbm.at[sidx.at[b]], bufs[b], sems[b])
            else:
                pltpu.async_copy(x_hbm.at[pl.ds(base + j * _CB, _CB)],
                                 bufs[b], sems[b])

        def drain_scatter(j, b):
            if gather:
                pltpu.make_async_copy(x_hbm.at[sidx.at[b]], bufs[b],
                                      sems[b]).wait()
            else:
                pltpu.make_async_copy(x_hbm.at[pl.ds(base + j * _CB, _CB)],
                                      bufs[b], sems[b]).wait()
            pltpu.sync_copy(bufs[b], agg_sh.at[ridx.at[b]], add=True)

        for b in range(2):
            prep(b, b)
        npair = (nch - 2) // 2

        def body(i, carry):
            for b in range(2):
                j = 2 * i + b
                drain_scatter(j, b)
                prep(j + 2, b)
            return carry

        lax.fori_loop(0, npair, body, 0)
        for j in range(2 * npair, nch):
            drain_scatter(j, j % 2)
            if j + 2 < nch:
                prep(j + 2, j % 2)

        plsc.subcore_barrier()
        pltpu.sync_copy(agg_sh.at[pl.ds(sid * rpt, rpt)],
                        out_hbm.at[cid, pl.ds(sid * rpt, rpt)])

    return k


def _full(shape):
    return pl.BlockSpec(shape, lambda i: tuple(0 for _ in shape))


def _enc_nodes(nodes, w0, b0, w1, b1, wmt_w):
    n, df = nodes.shape
    bn = 1000

    def body(x_ref, w0r, b0r, w1r, b1r, wmtr, hn_ref, m_ref):
        h = jnp.maximum(x_ref[...] @ w0r[...] + b0r[...], 0.0)
        hn = h @ w1r[...] + b1r[...]
        hn_ref[...] = hn
        m_ref[...] = hn @ wmtr[...]

    return pl.pallas_call(
        body,
        grid=(n // bn,),
        in_specs=[
            pl.BlockSpec((bn, df), lambda i: (i, 0)),
            _full((df, 64)), _full((1, 64)), _full((64, _D)), _full((1, _D)),
            _full((_D, _DW)),
        ],
        out_specs=[pl.BlockSpec((bn, _D), lambda i: (i, 0)),
                   pl.BlockSpec((bn, _DW), lambda i: (i, 0))],
        out_shape=[jax.ShapeDtypeStruct((n, _D), jnp.float32),
                   jax.ShapeDtypeStruct((n, _DW), jnp.float32)],
    )(nodes, w0, b0.reshape(1, 64), w1, b1.reshape(1, _D), wmt_w)


def _enc_edges(edges, w0, b0, w1, b1, wmb_w):
    e, de = edges.shape
    be = 4000

    def body(x_ref, w0r, b0r, w1r, b1r, wmbr, out_ref):
        h = jnp.maximum(x_ref[...] @ w0r[...] + b0r[...], 0.0)
        wc = w1r[...] @ wmbr[...]
        bc = b1r[...] @ wmbr[...]
        out_ref[...] = h @ wc + bc

    return pl.pallas_call(
        body,
        grid=(e // be,),
        in_specs=[
            pl.BlockSpec((be, de), lambda i: (i, 0)),
            _full((de, 64)), _full((1, 64)), _full((64, _D)), _full((1, _D)),
            _full((_D, _DW)),
        ],
        out_specs=pl.BlockSpec((be, _DW), lambda i: (i, 0)),
        out_shape=jax.ShapeDtypeStruct((e, _DW), jnp.float32),
    )(edges, w0, b0.reshape(1, 64), w1, b1.reshape(1, _D), wmb_w)


def _step(hn, parts, agge, w0a, w0b, b0, w1, b1, wn, lns, lnb, wmt_w, last,
          dw0, db0, dw1, db1):
    n, _ = hn.shape
    bn = 1000
    df = dw1.shape[1]

    def node_update(hn_ref, p_ref, pe_ref, w0ar, w0br, b0r, w1r, b1r, wnr,
                    sr, br):
        agg = (p_ref[0, :, :_D] + p_ref[1, :, :_D]
               + pe_ref[0, :, :_D] + pe_ref[1, :, :_D])
        t = jnp.maximum(hn_ref[...] @ w0ar[...] + agg @ w0br[...] + b0r[...],
                        0.0) @ w1r[...] + b1r[...]
        x = hn_ref[...] @ wnr[...] + t
        mu = jnp.mean(x, axis=-1, keepdims=True)
        var = jnp.mean((x - mu) ** 2, axis=-1, keepdims=True)
        return (x - mu) * lax.rsqrt(var + 1e-6) * sr[...] + br[...]

    common_specs = [
        pl.BlockSpec((bn, _D), lambda i: (i, 0)),
        pl.BlockSpec((_NC, bn, _DW), lambda i: (0, i, 0)),
        pl.BlockSpec((_NC, bn, _DW), lambda i: (0, i, 0)),
        _full((_D, _D)), _full((_D, _D)), _full((1, _D)), _full((_D, _D)),
        _full((1, _D)), _full((_D, _D)), _full((1, _D)), _full((1, _D)),
    ]
    common_args = (hn, parts, agge, w0a, w0b, b0.reshape(1, _D), w1,
                   b1.reshape(1, _D), wn, lns.reshape(1, _D),
                   lnb.reshape(1, _D))

    if not last:
        def body(hn_ref, p_ref, pe_ref, w0ar, w0br, b0r, w1r, b1r, wnr, sr,
                 br, wmtr, hn_out, m_out):
            y = node_update(hn_ref, p_ref, pe_ref, w0ar, w0br, b0r, w1r, b1r,
                            wnr, sr, br)
            hn_out[...] = y
            m_out[...] = y @ wmtr[...]

        return pl.pallas_call(
            body,
            grid=(n // bn,),
            in_specs=common_specs + [_full((_D, _DW))],
            out_specs=[pl.BlockSpec((bn, _D), lambda i: (i, 0)),
                       pl.BlockSpec((bn, _DW), lambda i: (i, 0))],
            out_shape=[jax.ShapeDtypeStruct((n, _D), jnp.float32),
                       jax.ShapeDtypeStruct((n, _DW), jnp.float32)],
        )(*common_args, wmt_w)

    def body(hn_ref, p_ref, pe_ref, w0ar, w0br, b0r, w1r, b1r, wnr, sr, br,
             dw0r, db0r, dw1r, db1r, out_ref):
        y = node_update(hn_ref, p_ref, pe_ref, w0ar, w0br, b0r, w1r, b1r,
                        wnr, sr, br)
        d = jnp.maximum(y @ dw0r[...] + db0r[...], 0.0) @ dw1r[...] + db1r[...]
        out_ref[...] = d

    return pl.pallas_call(
        body,
        grid=(n // bn,),
        in_specs=common_specs + [_full((_D, 64)), _full((1, 64)),
                                 _full((64, df)), _full((1, df))],
        out_specs=pl.BlockSpec((bn, df), lambda i: (i, 0)),
        out_shape=jax.ShapeDtypeStruct((n, df), jnp.float32),
    )(*common_args, dw0, db0.reshape(1, 64), dw1, db1.reshape(1, df))


def kernel(nodes, edges, senders, receivers,
           enc_node_W0, enc_node_b0, enc_node_W1, enc_node_b1,
           enc_edge_W0, enc_edge_b0, enc_edge_W1, enc_edge_b1,
           W_message, W_node,
           nodeMLP_W0, nodeMLP_b0, nodeMLP_W1, nodeMLP_b1,
           ln_scale, ln_bias,
           dec_W0, dec_b0, dec_W1, dec_b1):
    n, _ = nodes.shape
    e, _ = edges.shape
    epw = e // _NW
    nch = epw // _CB

    senders = senders.astype(jnp.int32)
    receivers = receivers.astype(jnp.int32)
    packed = ((receivers << 14) | senders).reshape(_NW, nch, _CB)
    recv3 = receivers.reshape(_NW, nch, _CB)
    zeros = jnp.zeros((_NP // _NS, _DW), jnp.float32)

    wmt_w = jnp.pad(W_message[:_D], ((0, 0), (0, _DW - _D)))
    wmb_w = jnp.pad(W_message[_D:], ((0, 0), (0, _DW - _D)))
    w0a = nodeMLP_W0[:_D]
    w0b = nodeMLP_W0[_D:]

    gather_segsum = _build_segsum(e, gather=True)
    linear_segsum = _build_segsum(e, gather=False)

    hn, m = _enc_nodes(nodes, enc_node_W0, enc_node_b0, enc_node_W1,
                       enc_node_b1, wmt_w)
    msg_e = _enc_edges(edges, enc_edge_W0, enc_edge_b0, enc_edge_W1,
                       enc_edge_b1, wmb_w)
    agge = linear_segsum(msg_e, recv3, zeros)

    for step in range(5):
        parts = gather_segsum(m, packed, zeros)
        out = _step(hn, parts, agge, w0a, w0b, nodeMLP_b0, nodeMLP_W1,
                    nodeMLP_b1, W_node, ln_scale, ln_bias, wmt_w,
                    step == 4, dec_W0, dec_b0, dec_W1, dec_b1)
        if step < 4:
            hn, m = out
        else:
            return out


# speedup vs baseline: 7.8275x; 1.1020x over previous
import functools

import jax
import jax.numpy as jnp
from jax import lax
from jax.experimental import pallas as pl
from jax.experimental.pallas import tpu as pltpu
from jax.experimental.pallas import tpu_sc as plsc

_NC = 2
_NS = 16
_NW = _NC * _NS
_CB = 80
_D = 32
_DW = 128
_NP = 10112


def _sc_mesh():
    return plsc.VectorSubcoreMesh(core_axis_name="c", subcore_axis_name="s")


def _build_segsum(n_edges, gather):
    epw = n_edges // _NW
    nch = epw // _CB
    rpt = _NP // _NS

    scratch = [
        pltpu.VMEM((nch, _CB), jnp.int32),
        pltpu.VMEM((3, _CB), jnp.int32),
        pltpu.VMEM((3, _CB), jnp.int32),
        pltpu.VMEM((_CB, _DW), jnp.float32),
        pltpu.VMEM((_CB, _DW), jnp.float32),
        pltpu.VMEM((_CB, _DW), jnp.float32),
        pltpu.VMEM_SHARED((_NP, _DW), jnp.float32),
        pltpu.SemaphoreType.DMA,
        pltpu.SemaphoreType.DMA,
        pltpu.SemaphoreType.DMA,
        pltpu.SemaphoreType.DMA,
        pltpu.SemaphoreType.DMA,
        pltpu.SemaphoreType.DMA,
    ]

    @functools.partial(
        pl.kernel,
        mesh=_sc_mesh(),
        out_type=jax.ShapeDtypeStruct((_NC, _NP, _DW), jnp.float32),
        scratch_types=scratch,
    )
    def k(x_hbm, idx_hbm, z_hbm, out_hbm, staged, sidx, ridx,
          buf0, buf1, buf2, agg_sh, g0, g1, g2, s0, s1, s2):
        cid = lax.axis_index("c")
        sid = lax.axis_index("s")
        wid = sid * _NC + cid
        base = wid * epw
        pltpu.sync_copy(z_hbm, agg_sh.at[pl.ds(sid * rpt, rpt)])
        pltpu.sync_copy(idx_hbm.at[wid], staged)
        plsc.subcore_barrier()

        bufs = (buf0, buf1, buf2)
        gsem = (g0, g1, g2)
        ssem = (s0, s1, s2)

        def gsrc(j, b):
            if gather:
                return x_hbm.at[sidx.at[b]]
            return x_hbm.at[pl.ds(base + j * _CB, _CB)]

        def prep(j, b):
            for t in range(_CB // 16):
                sl = pl.ds(t * 16, 16)
                p = staged[j, sl]
                if gather:
                    sidx[b, sl] = lax.bitwise_and(p, 16383)
                    ridx[b, sl] = lax.shift_right_logical(p, 14)
                else:
                    ridx[b, sl] = p
            pltpu.async_copy(gsrc(j, b), bufs[b], gsem[b])

        def wait_gather(j, b):
            pltpu.make_async_copy(gsrc(j, b), bufs[b], gsem[b]).wait()

        def scatter(b):
            pltpu.async_copy(bufs[b], agg_sh.at[ridx.at[b]], ssem[b],
                             add=True)

        def wait_scatter(b):
            pltpu.make_async_copy(bufs[b], agg_sh.at[ridx.at[b]],
                                  ssem[b]).wait()

        prep(0, 0)
        for j in range(2):
            prep(j + 1, (j + 1) % 3)
            wait_gather(j, j % 3)
            scatter(j % 3)

        start = 2
        main = ((nch - 1) - start) // 3 * 3

        def bodyfn(i, carry):
            for t in range(3):
                j = start + 3 * i + t
                b0 = (start + t) % 3
                b1 = (start + t + 1) % 3
                wait_scatter(b1)
                prep(j + 1, b1)
                wait_gather(j, b0)
                scatter(b0)
            return carry

        lax.fori_loop(0, main // 3, bodyfn, 0)
        for j in range(start + main, nch):
            b0 = j % 3
            b1 = (j + 1) % 3
            wait_scatter(b1)
            if j + 1 < nch:
                prep(j + 1, b1)
            wait_gather(j, b0)
            scatter(b0)
        wait_scatter((nch - 2) % 3)
        wait_scatter((nch - 1) % 3)

        plsc.subcore_barrier()
        pltpu.sync_copy(agg_sh.at[pl.ds(sid * rpt, rpt)],
                        out_hbm.at[cid, pl.ds(sid * rpt, rpt)])

    return k


def _full(shape):
    return pl.BlockSpec(shape, lambda i: tuple(0 for _ in shape))


def _enc_nodes(nodes, w0, b0, w1, b1, wmt_w):
    n, df = nodes.shape
    bn = 1000

    def body(x_ref, w0r, b0r, w1r, b1r, wmtr, hn_ref, m_ref):
        h = jnp.maximum(x_ref[...] @ w0r[...] + b0r[...], 0.0)
        hn = h @ w1r[...] + b1r[...]
        hn_ref[...] = hn
        m_ref[...] = hn @ wmtr[...]

    return pl.pallas_call(
        body,
        grid=(n // bn,),
        in_specs=[
            pl.BlockSpec((bn, df), lambda i: (i, 0)),
            _full((df, 64)), _full((1, 64)), _full((64, _D)), _full((1, _D)),
            _full((_D, _DW)),
        ],
        out_specs=[pl.BlockSpec((bn, _D), lambda i: (i, 0)),
                   pl.BlockSpec((bn, _DW), lambda i: (i, 0))],
        out_shape=[jax.ShapeDtypeStruct((n, _D), jnp.float32),
                   jax.ShapeDtypeStruct((n, _DW), jnp.float32)],
    )(nodes, w0, b0.reshape(1, 64), w1, b1.reshape(1, _D), wmt_w)


def _enc_edges(edges, w0, b0, w1, b1, wmb_w):
    e, de = edges.shape
    be = 4000

    def body(x_ref, w0r, b0r, w1r, b1r, wmbr, out_ref):
        h = jnp.maximum(x_ref[...] @ w0r[...] + b0r[...], 0.0)
        wc = w1r[...] @ wmbr[...]
        bc = b1r[...] @ wmbr[...]
        out_ref[...] = h @ wc + bc

    return pl.pallas_call(
        body,
        grid=(e // be,),
        in_specs=[
            pl.BlockSpec((be, de), lambda i: (i, 0)),
            _full((de, 64)), _full((1, 64)), _full((64, _D)), _full((1, _D)),
            _full((_D, _DW)),
        ],
        out_specs=pl.BlockSpec((be, _DW), lambda i: (i, 0)),
        out_shape=jax.ShapeDtypeStruct((e, _DW), jnp.float32),
    )(edges, w0, b0.reshape(1, 64), w1, b1.reshape(1, _D), wmb_w)


def _step(hn, parts, agge, w0a, w0b, b0, w1, b1, wn, lns, lnb, wmt_w, last,
          dw0, db0, dw1, db1):
    n, _ = hn.shape
    bn = 1000
    df = dw1.shape[1]

    def node_update(hn_ref, p_ref, pe_ref, w0ar, w0br, b0r, w1r, b1r, wnr,
                    sr, br):
        agg = (p_ref[0, :, :_D] + p_ref[1, :, :_D]
               + pe_ref[0, :, :_D] + pe_ref[1, :, :_D])
        t = jnp.maximum(hn_ref[...] @ w0ar[...] + agg @ w0br[...] + b0r[...],
                        0.0) @ w1r[...] + b1r[...]
        x = hn_ref[...] @ wnr[...] + t
        mu = jnp.mean(x, axis=-1, keepdims=True)
        var = jnp.mean((x - mu) ** 2, axis=-1, keepdims=True)
        return (x - mu) * lax.rsqrt(var + 1e-6) * sr[...] + br[...]

    common_specs = [
        pl.BlockSpec((bn, _D), lambda i: (i, 0)),
        pl.BlockSpec((_NC, bn, _DW), lambda i: (0, i, 0)),
        pl.BlockSpec((_NC, bn, _DW), lambda i: (0, i, 0)),
        _full((_D, _D)), _full((_D, _D)), _full((1, _D)), _full((_D, _D)),
        _full((1, _D)), _full((_D, _D)), _full((1, _D)), _full((1, _D)),
    ]
    common_args = (hn, parts, agge, w0a, w0b, b0.reshape(1, _D), w1,
                   b1.reshape(1, _D), wn, lns.reshape(1, _D),
                   lnb.reshape(1, _D))

    if not last:
        def body(hn_ref, p_ref, pe_ref, w0ar, w0br, b0r, w1r, b1r, wnr, sr,
                 br, wmtr, hn_out, m_out):
            y = node_update(hn_ref, p_ref, pe_ref, w0ar, w0br, b0r, w1r, b1r,
                            wnr, sr, br)
            hn_out[...] = y
            m_out[...] = y @ wmtr[...]

        return pl.pallas_call(
            body,
            grid=(n // bn,),
            in_specs=common_specs + [_full((_D, _DW))],
            out_specs=[pl.BlockSpec((bn, _D), lambda i: (i, 0)),
                       pl.BlockSpec((bn, _DW), lambda i: (i, 0))],
            out_shape=[jax.ShapeDtypeStruct((n, _D), jnp.float32),
                       jax.ShapeDtypeStruct((n, _DW), jnp.float32)],
        )(*common_args, wmt_w)

    def body(hn_ref, p_ref, pe_ref, w0ar, w0br, b0r, w1r, b1r, wnr, sr, br,
             dw0r, db0r, dw1r, db1r, out_ref):
        y = node_update(hn_ref, p_ref, pe_ref, w0ar, w0br, b0r, w1r, b1r,
                        wnr, sr, br)
        d = jnp.maximum(y @ dw0r[...] + db0r[...], 0.0) @ dw1r[...] + db1r[...]
        out_ref[...] = d

    return pl.pallas_call(
        body,
        grid=(n // bn,),
        in_specs=common_specs + [_full((_D, 64)), _full((1, 64)),
                                 _full((64, df)), _full((1, df))],
        out_specs=pl.BlockSpec((bn, df), lambda i: (i, 0)),
        out_shape=jax.ShapeDtypeStruct((n, df), jnp.float32),
    )(*common_args, dw0, db0.reshape(1, 64), dw1, db1.reshape(1, df))


def kernel(nodes, edges, senders, receivers,
           enc_node_W0, enc_node_b0, enc_node_W1, enc_node_b1,
           enc_edge_W0, enc_edge_b0, enc_edge_W1, enc_edge_b1,
           W_message, W_node,
           nodeMLP_W0, nodeMLP_b0, nodeMLP_W1, nodeMLP_b1,
           ln_scale, ln_bias,
           dec_W0, dec_b0, dec_W1, dec_b1):
    n, _ = nodes.shape
    e, _ = edges.shape
    epw = e // _NW
    nch = epw // _CB

    senders = senders.astype(jnp.int32)
    receivers = receivers.astype(jnp.int32)
    packed = ((receivers << 14) | senders).reshape(_NW, nch, _CB)
    recv3 = receivers.reshape(_NW, nch, _CB)
    zeros = jnp.zeros((_NP // _NS, _DW), jnp.float32)

    wmt_w = jnp.pad(W_message[:_D], ((0, 0), (0, _DW - _D)))
    wmb_w = jnp.pad(W_message[_D:], ((0, 0), (0, _DW - _D)))
    w0a = nodeMLP_W0[:_D]
    w0b = nodeMLP_W0[_D:]

    gather_segsum = _build_segsum(e, gather=True)
    linear_segsum = _build_segsum(e, gather=False)

    hn, m = _enc_nodes(nodes, enc_node_W0, enc_node_b0, enc_node_W1,
                       enc_node_b1, wmt_w)
    msg_e = _enc_edges(edges, enc_edge_W0, enc_edge_b0, enc_edge_W1,
                       enc_edge_b1, wmb_w)
    agge = linear_segsum(msg_e, recv3, zeros)

    for step in range(5):
        parts = gather_segsum(m, packed, zeros)
        out = _step(hn, parts, agge, w0a, w0b, nodeMLP_b0, nodeMLP_W1,
                    nodeMLP_b1, W_node, ln_scale, ln_bias, wmt_w,
                    step == 4, dec_W0, dec_b0, dec_W1, dec_b1)
        if step < 4:
            hn, m = out
        else:
            return out
